# Initial kernel scaffold; baseline (speedup 1.0000x reference)
import jax
import jax.numpy as jnp
from jax.experimental import pallas as pl


def kernel(pred, target):
    raise NotImplementedError("write your pallas kernel here")


# speedup vs baseline: 26.9955x; 26.9955x over previous
import functools

import jax
import jax.numpy as jnp
from jax import lax
from jax.experimental import pallas as pl
from jax.experimental.pallas import tpu as pltpu
from jax.experimental.pallas import tpu_sc as plsc

_ALPHA = 0.25
_LOSS_WEIGHT = 1.0

_NC = 2
_NS = 16
_NW = _NC * _NS
_LANES = 16


def _focal_vec(p, tgt):
    t = tgt == 0
    ap = jnp.abs(p)
    e = jnp.exp(-ap)
    r = 1.0 / (1.0 + e)
    s = jnp.where(p >= 0, r, e * r)
    z = e / (e + 2.0)
    w = z * z
    poly = 1.0 + w * (1.0 / 3.0 + w * (1.0 / 5.0 + w * (1.0 / 7.0 + w * (1.0 / 9.0))))
    l1p = 2.0 * z * poly
    q = jnp.where(t, -p, p)
    bce = jnp.maximum(q, 0.0) + l1p
    pt = jnp.where(t, 1.0 - s, s)
    af = jnp.where(t, _ALPHA, 1.0 - _ALPHA)
    return bce * af * pt * pt


def kernel(pred, target):
    n = pred.shape[0]
    per_w = n // _NW
    vecs = per_w // _LANES
    predf = pred.reshape(n)
    mesh = plsc.VectorSubcoreMesh(core_axis_name="c", subcore_axis_name="s")

    @functools.partial(
        pl.kernel,
        mesh=mesh,
        out_type=jax.ShapeDtypeStruct((_NW, _LANES), jnp.float32),
        scratch_types=[
            pltpu.VMEM((per_w,), jnp.float32),
            pltpu.VMEM((per_w,), jnp.int32),
            pltpu.VMEM((_LANES,), jnp.float32),
        ],
    )
    def sc_loss(pred_hbm, tgt_hbm, out_hbm, pred_v, tgt_v, acc_v):
        wid = lax.axis_index("s") * _NC + lax.axis_index("c")
        base = wid * per_w
        pltpu.sync_copy(pred_hbm.at[pl.ds(base, per_w)], pred_v)
        pltpu.sync_copy(tgt_hbm.at[pl.ds(base, per_w)], tgt_v)

        def body(i, acc):
            p = pred_v[pl.ds(i * _LANES, _LANES)]
            tg = tgt_v[pl.ds(i * _LANES, _LANES)]
            return acc + _focal_vec(p, tg)

        acc = lax.fori_loop(0, vecs, body, jnp.zeros((_LANES,), jnp.float32))
        acc_v[...] = acc
        pltpu.sync_copy(acc_v, out_hbm.at[wid])

    partials = sc_loss(predf, target)
    return _LOSS_WEIGHT * (jnp.sum(partials) / n)


# speedup vs baseline: 27.3001x; 1.0113x over previous
import functools

import jax
import jax.numpy as jnp
from jax import lax
from jax.experimental import pallas as pl
from jax.experimental.pallas import tpu as pltpu
from jax.experimental.pallas import tpu_sc as plsc

_ALPHA = 0.25
_LOSS_WEIGHT = 1.0

_NC = 2
_NS = 16
_NW = _NC * _NS
_LANES = 16


def _focal_vec(p, tgt):
    t = tgt == 0
    nonneg = p >= 0
    ap = jnp.abs(p)
    e = jnp.exp(-ap)
    r = 1.0 / (1.0 + e)
    er = e * r
    z = e / (e + 2.0)
    w = z * z
    poly = 1.0 + w * (1.0 / 3.0 + w * (1.0 / 5.0 + w * (1.0 / 7.0 + w * (1.0 / 9.0))))
    l1p = 2.0 * z * poly
    q = jnp.where(t, -p, p)
    bce = jnp.maximum(q, 0.0) + l1p
    s = jnp.where(nonneg, r, er)
    pt = jnp.where(t, 1.0 - s, s)
    af = jnp.where(t, _ALPHA, 1.0 - _ALPHA)
    return bce * af * pt * pt


def kernel(pred, target):
    n = pred.shape[0]
    per_w = n // _NW
    vecs = per_w // _LANES
    predf = pred.reshape(n)
    mesh = plsc.VectorSubcoreMesh(core_axis_name="c", subcore_axis_name="s")

    @functools.partial(
        pl.kernel,
        mesh=mesh,
        out_type=jax.ShapeDtypeStruct((_NW, _LANES), jnp.float32),
        scratch_types=[
            pltpu.VMEM((per_w,), jnp.float32),
            pltpu.VMEM((per_w,), jnp.int32),
            pltpu.VMEM((_LANES,), jnp.float32),
            pltpu.SemaphoreType.DMA,
            pltpu.SemaphoreType.DMA,
        ],
    )
    def sc_loss(pred_hbm, tgt_hbm, out_hbm, pred_v, tgt_v, acc_v, sem_p, sem_t):
        wid = lax.axis_index("s") * _NC + lax.axis_index("c")
        base = wid * per_w
        cp_p = pltpu.async_copy(pred_hbm.at[pl.ds(base, per_w)], pred_v, sem_p)
        cp_t = pltpu.async_copy(tgt_hbm.at[pl.ds(base, per_w)], tgt_v, sem_t)
        cp_p.wait()
        cp_t.wait()

        unroll = 4
        zero = jnp.zeros((_LANES,), jnp.float32)

        def body(i, accs):
            b = i * (unroll * _LANES)
            out = []
            for k in range(unroll):
                p = pred_v[pl.ds(b + k * _LANES, _LANES)]
                tg = tgt_v[pl.ds(b + k * _LANES, _LANES)]
                out.append(accs[k] + _focal_vec(p, tg))
            return tuple(out)

        accs = lax.fori_loop(0, vecs // unroll, body, (zero,) * unroll)
        acc_v[...] = (accs[0] + accs[1]) + (accs[2] + accs[3])
        pltpu.sync_copy(acc_v, out_hbm.at[wid])

    partials = sc_loss(predf, target)
    return _LOSS_WEIGHT * (jnp.sum(partials) / n)
